# Initial kernel scaffold; baseline (speedup 1.0000x reference)
import jax
import jax.numpy as jnp
from jax.experimental import pallas as pl


def kernel(x, gate_W, gate_b, up_proj, gate_proj, down_proj):
    raise NotImplementedError("write your pallas kernel here")


# speedup vs baseline: 3.5994x; 3.5994x over previous
import jax
import jax.numpy as jnp
from jax.experimental import pallas as pl
from jax.experimental.pallas import tpu as pltpu


def _moe_body(x_ref, gw_ref, gb_ref, up_ref, gp_ref, dp_ref, out_ref, idx_ref):
    e = pl.program_id(0)
    n_e = gw_ref.shape[0]

    @pl.when(e == 0)
    def _router():
        x = x_ref[...]
        logits = jax.lax.dot_general(
            x, gw_ref[...], (((1,), (1,)), ((), ())),
            preferred_element_type=jnp.float32)
        logits = logits + gb_ref[...]
        m = jnp.max(logits, axis=1, keepdims=True)
        p = jnp.exp(logits - m)
        p = p / jnp.sum(p, axis=1, keepdims=True)
        pm = jnp.max(p, axis=1, keepdims=True)
        ids = jax.lax.broadcasted_iota(jnp.int32, p.shape, 1)
        idx_ref[...] = jnp.min(
            jnp.where(p >= pm, ids, n_e), axis=1, keepdims=True)
        out_ref[...] = jnp.zeros_like(out_ref)

    mask = idx_ref[...] == e
    x = jnp.where(mask, x_ref[...], 0.0)
    g = jax.lax.dot_general(x, gp_ref[0], (((1,), (1,)), ((), ())),
                            preferred_element_type=jnp.float32)
    u = jax.lax.dot_general(x, up_ref[0], (((1,), (1,)), ((), ())),
                            preferred_element_type=jnp.float32)
    h = (g * jax.nn.sigmoid(g)) * u
    y = jax.lax.dot_general(h, dp_ref[0], (((1,), (1,)), ((), ())),
                            preferred_element_type=jnp.float32)
    out_ref[...] += y


def kernel(x, gate_W, gate_b, up_proj, gate_proj, down_proj):
    B, S, D = x.shape
    T = B * S
    E, H, _ = up_proj.shape
    xf = x.reshape(T, D)

    out = pl.pallas_call(
        _moe_body,
        grid=(E,),
        in_specs=[
            pl.BlockSpec((T, D), lambda e: (0, 0)),
            pl.BlockSpec((E, D), lambda e: (0, 0)),
            pl.BlockSpec((1, E), lambda e: (0, 0)),
            pl.BlockSpec((1, H, D), lambda e: (e, 0, 0)),
            pl.BlockSpec((1, H, D), lambda e: (e, 0, 0)),
            pl.BlockSpec((1, D, H), lambda e: (e, 0, 0)),
        ],
        out_specs=pl.BlockSpec((T, D), lambda e: (0, 0)),
        out_shape=jax.ShapeDtypeStruct((T, D), x.dtype),
        scratch_shapes=[pltpu.VMEM((T, 1), jnp.int32)],
        compiler_params=pltpu.CompilerParams(
            dimension_semantics=("arbitrary",)),
    )(xf, gate_W, gate_b.reshape(1, E), up_proj, gate_proj, down_proj)
    return out.reshape(B, S, D)


# speedup vs baseline: 3.6386x; 1.0109x over previous
import jax
import jax.numpy as jnp
from jax.experimental import pallas as pl
from jax.experimental.pallas import tpu as pltpu


def _moe_body(x_ref, gw_ref, gb_ref, up_a, up_b, gp_a, gp_b, dp_a, dp_b,
              out_ref, idx_ref):
    e = pl.program_id(0)
    n_e = gw_ref.shape[0]

    @pl.when(e == 0)
    def _router():
        x = x_ref[...]
        logits = jax.lax.dot_general(
            x, gw_ref[...], (((1,), (1,)), ((), ())),
            preferred_element_type=jnp.float32)
        logits = logits + gb_ref[...]
        m = jnp.max(logits, axis=1, keepdims=True)
        p = jnp.exp(logits - m)
        p = p / jnp.sum(p, axis=1, keepdims=True)
        pm = jnp.max(p, axis=1, keepdims=True)
        ids = jax.lax.broadcasted_iota(jnp.int32, p.shape, 1)
        idx_ref[...] = jnp.min(
            jnp.where(p >= pm, ids, n_e), axis=1, keepdims=True)
        out_ref[...] = jnp.zeros_like(out_ref)

    cdims = (((1,), (1,)), ((), ()))
    mask = idx_ref[...] == e
    x = jnp.where(mask, x_ref[...], 0.0)
    y = jnp.zeros_like(out_ref)
    for gp_h, up_h, dp_h in ((gp_a, up_a, dp_a), (gp_b, up_b, dp_b)):
        g = jax.lax.dot_general(x, gp_h[0], cdims,
                                preferred_element_type=jnp.float32)
        u = jax.lax.dot_general(x, up_h[0], cdims,
                                preferred_element_type=jnp.float32)
        h = (g * jax.nn.sigmoid(g)) * u
        y = y + jax.lax.dot_general(h, dp_h[0], cdims,
                                    preferred_element_type=jnp.float32)
    out_ref[...] += y


def kernel(x, gate_W, gate_b, up_proj, gate_proj, down_proj):
    B, S, D = x.shape
    T = B * S
    E, H, _ = up_proj.shape
    Hh = H // 2
    xf = x.reshape(T, D)

    hd_a = pl.BlockSpec((1, Hh, D), lambda e: (e, 0, 0))
    hd_b = pl.BlockSpec((1, Hh, D), lambda e: (e, 1, 0))
    dh_a = pl.BlockSpec((1, D, Hh), lambda e: (e, 0, 0))
    dh_b = pl.BlockSpec((1, D, Hh), lambda e: (e, 0, 1))

    out = pl.pallas_call(
        _moe_body,
        grid=(E,),
        in_specs=[
            pl.BlockSpec((T, D), lambda e: (0, 0)),
            pl.BlockSpec((E, D), lambda e: (0, 0)),
            pl.BlockSpec((1, E), lambda e: (0, 0)),
            hd_a, hd_b, hd_a, hd_b, dh_a, dh_b,
        ],
        out_specs=pl.BlockSpec((T, D), lambda e: (0, 0)),
        out_shape=jax.ShapeDtypeStruct((T, D), x.dtype),
        scratch_shapes=[pltpu.VMEM((T, 1), jnp.int32)],
        compiler_params=pltpu.CompilerParams(
            dimension_semantics=("arbitrary",)),
    )(xf, gate_W, gate_b.reshape(1, E),
      up_proj, up_proj, gate_proj, gate_proj, down_proj, down_proj)
    return out.reshape(B, S, D)


# speedup vs baseline: 3.8948x; 1.0704x over previous
import jax
import jax.numpy as jnp
from jax.experimental import pallas as pl
from jax.experimental.pallas import tpu as pltpu

def _router_body(x_ref, gw_ref, gb_ref, idx_ref, order_ref, nact_ref):
    n_e = gw_ref.shape[0]
    x = x_ref[...]
    logits = jax.lax.dot_general(
        x, gw_ref[...], (((1,), (1,)), ((), ())),
        preferred_element_type=jnp.float32)
    logits = logits + gb_ref[...]
    m = jnp.max(logits, axis=1, keepdims=True)
    p = jnp.exp(logits - m)
    p = p / jnp.sum(p, axis=1, keepdims=True)
    pm = jnp.max(p, axis=1, keepdims=True)
    ids = jax.lax.broadcasted_iota(jnp.int32, p.shape, 1)
    idx = jnp.min(jnp.where(p >= pm, ids, n_e), axis=1, keepdims=True)
    idx_ref[...] = idx

    iota_e = jax.lax.broadcasted_iota(jnp.int32, (1, n_e), 1)
    onehot = idx == iota_e
    active = jnp.any(onehot, axis=0, keepdims=True)
    act_f = active.astype(jnp.float32)
    r = jax.lax.broadcasted_iota(jnp.int32, (n_e, n_e), 0)
    c = jax.lax.broadcasted_iota(jnp.int32, (n_e, n_e), 1)
    lt = (r < c).astype(jnp.float32)
    pos = jax.lax.dot_general(act_f, lt, (((1,), (0,)), ((), ())),
                              preferred_element_type=jnp.float32)
    pos = pos.astype(jnp.int32)
    nact_ref[...] = jnp.sum(active.astype(jnp.int32), axis=1, keepdims=True)
    posb = jnp.broadcast_to(pos, (n_e, n_e))
    actb = jnp.broadcast_to(active, (n_e, n_e))
    cmp = (posb == r) & actb
    order_raw = jnp.min(jnp.where(cmp, c, n_e), axis=1, keepdims=True)
    last_active = jnp.max(jnp.where(active, iota_e, -1))
    order_ref[...] = jnp.where(order_raw == n_e, last_active, order_raw)


def _expert_body(order_sm, nact_sm, idx_ref, x_ref, up_a, up_b, gp_a, gp_b,
                 dp_a, dp_b, out_ref):
    i = pl.program_id(0)

    @pl.when(i == 0)
    def _init():
        out_ref[...] = jnp.zeros_like(out_ref)

    @pl.when(i < nact_sm[0])
    def _compute():
        e = order_sm[i]
        cdims = (((1,), (1,)), ((), ()))
        mask = idx_ref[...] == e
        x = jnp.where(mask, x_ref[...], 0.0)
        y = jnp.zeros_like(out_ref)
        for gp_h, up_h, dp_h in ((gp_a, up_a, dp_a), (gp_b, up_b, dp_b)):
            g = jax.lax.dot_general(x, gp_h[0], cdims,
                                    preferred_element_type=jnp.float32)
            u = jax.lax.dot_general(x, up_h[0], cdims,
                                    preferred_element_type=jnp.float32)
            h = (g * jax.nn.sigmoid(g)) * u
            y = y + jax.lax.dot_general(h, dp_h[0], cdims,
                                        preferred_element_type=jnp.float32)
        out_ref[...] += y


def kernel(x, gate_W, gate_b, up_proj, gate_proj, down_proj):
    B, S, D = x.shape
    T = B * S
    E, H, _ = up_proj.shape
    Hh = H // 2
    xf = x.reshape(T, D)

    idx, order, nact = pl.pallas_call(
        _router_body,
        grid=(1,),
        in_specs=[
            pl.BlockSpec((T, D), lambda i: (0, 0)),
            pl.BlockSpec((E, D), lambda i: (0, 0)),
            pl.BlockSpec((1, E), lambda i: (0, 0)),
        ],
        out_specs=[
            pl.BlockSpec((T, 1), lambda i: (0, 0)),
            pl.BlockSpec((E, 1), lambda i: (0, 0)),
            pl.BlockSpec((1, 1), lambda i: (0, 0)),
        ],
        out_shape=[
            jax.ShapeDtypeStruct((T, 1), jnp.int32),
            jax.ShapeDtypeStruct((E, 1), jnp.int32),
            jax.ShapeDtypeStruct((1, 1), jnp.int32),
        ],
    )(xf, gate_W, gate_b.reshape(1, E))

    hd_a = pl.BlockSpec((1, Hh, D), lambda i, o, n: (o[i], 0, 0))
    hd_b = pl.BlockSpec((1, Hh, D), lambda i, o, n: (o[i], 1, 0))
    dh_a = pl.BlockSpec((1, D, Hh), lambda i, o, n: (o[i], 0, 0))
    dh_b = pl.BlockSpec((1, D, Hh), lambda i, o, n: (o[i], 0, 1))

    grid_spec = pltpu.PrefetchScalarGridSpec(
        num_scalar_prefetch=2,
        grid=(E,),
        in_specs=[
            pl.BlockSpec((T, 1), lambda i, o, n: (0, 0)),
            pl.BlockSpec((T, D), lambda i, o, n: (0, 0)),
            hd_a, hd_b, hd_a, hd_b, dh_a, dh_b,
        ],
        out_specs=pl.BlockSpec((T, D), lambda i, o, n: (0, 0)),
    )
    out = pl.pallas_call(
        _expert_body,
        grid_spec=grid_spec,
        out_shape=jax.ShapeDtypeStruct((T, D), x.dtype),
        compiler_params=pltpu.CompilerParams(
            dimension_semantics=("arbitrary",)),
    )(order.reshape(E), nact.reshape(1), idx, xf,
      up_proj, up_proj, gate_proj, gate_proj, down_proj, down_proj)
    return out.reshape(B, S, D)
